# baseline (device time: 34614 ns/iter reference)
import jax
import jax.numpy as jnp
from jax import lax
from jax.experimental import pallas as pl
from jax.experimental.pallas import tpu as pltpu

N_DEV = 32
SQ = 256
D = 1024
H = 8
DH = 128
C = SQ // N_DEV
SCALE = 0.08838834764831843


def _allreduce(partial_bf):
    m, n = partial_bf.shape
    bf = jnp.bfloat16
    f32 = jnp.float32

    def body(in_ref, out_ref, rs_rbuf, rs_send, rs_recv, ag_send, ag_recv):
        me = lax.axis_index("i")

        barrier_sem = pltpu.get_barrier_semaphore()
        for j in range(N_DEV - 1):
            pl.semaphore_signal(
                barrier_sem, inc=1,
                device_id=((me + 1 + j) & (N_DEV - 1),),
                device_id_type=pl.DeviceIdType.MESH,
            )
        pl.semaphore_wait(barrier_sem, N_DEV - 1)

        rs = []
        for j in range(N_DEV - 1):
            p = (me + 1 + j) & (N_DEV - 1)
            rdma = pltpu.make_async_remote_copy(
                src_ref=in_ref.at[pl.ds(pl.multiple_of(p * C, 8), C), :],
                dst_ref=rs_rbuf.at[j],
                send_sem=rs_send.at[j],
                recv_sem=rs_recv.at[j],
                device_id=(p,),
                device_id_type=pl.DeviceIdType.MESH,
            )
            rdma.start()
            rs.append(rdma)

        my_row = pl.multiple_of(me * C, 8)
        acc = in_ref[pl.ds(my_row, C), :].astype(f32)
        for j in range(N_DEV - 1):
            rs[j].wait()
            acc = acc + rs_rbuf[j].astype(f32)
        out_ref[pl.ds(my_row, C), :] = acc.astype(bf)

        ag = []
        for j in range(N_DEV - 1):
            p = (me + 1 + j) & (N_DEV - 1)
            rdma = pltpu.make_async_remote_copy(
                src_ref=out_ref.at[pl.ds(my_row, C), :],
                dst_ref=out_ref.at[pl.ds(my_row, C), :],
                send_sem=ag_send.at[j],
                recv_sem=ag_recv.at[j],
                device_id=(p,),
                device_id_type=pl.DeviceIdType.MESH,
            )
            rdma.start()
            ag.append(rdma)
        for rdma in ag:
            rdma.wait()

    return pl.pallas_call(
        body,
        out_shape=jax.ShapeDtypeStruct((m, n), bf),
        in_specs=[pl.BlockSpec(memory_space=pltpu.VMEM)],
        out_specs=pl.BlockSpec(memory_space=pltpu.VMEM),
        scratch_shapes=[
            pltpu.VMEM((N_DEV - 1, C, n), bf),
            pltpu.SemaphoreType.DMA((N_DEV - 1,)),
            pltpu.SemaphoreType.DMA((N_DEV - 1,)),
            pltpu.SemaphoreType.DMA((N_DEV - 1,)),
            pltpu.SemaphoreType.DMA((N_DEV - 1,)),
        ],
        compiler_params=pltpu.CompilerParams(collective_id=0),
    )(partial_bf)


def kernel(x, Wq, Wo, Wk, Wv):
    bf = jnp.bfloat16
    f32 = jnp.float32
    xb = x.reshape(SQ, D).astype(bf)
    q = jnp.dot(xb, Wq.astype(bf), preferred_element_type=bf)
    k = jnp.dot(xb, Wk.astype(bf), preferred_element_type=bf)
    v = jnp.dot(xb, Wv.astype(bf), preferred_element_type=bf)
    q = q.reshape(SQ, H, DH)
    k = k.reshape(SQ, H, DH)
    v = v.reshape(SQ, H, DH)
    s = jnp.einsum("ihd,jhd->hij", q, k, preferred_element_type=f32) * SCALE
    p = jax.nn.softmax(s, axis=-1).astype(bf)
    o = jnp.einsum("hij,jhd->ihd", p, v, preferred_element_type=bf)
    partial_bf = jnp.dot(
        o.reshape(SQ, H * DH), Wo.astype(bf), preferred_element_type=bf
    )
    out = _allreduce(partial_bf)
    return out.reshape(1, SQ, D)
